# baseline (speedup 1.0000x reference)
import functools
import math

import jax
import jax.numpy as jnp
from jax import lax
from jax.experimental import pallas as pl
from jax.experimental.pallas import tpu as pltpu
from jax.experimental.pallas import tpu_sc as plsc

_NC = 2
_NS = 16
_NW = _NC * _NS
_LANES = 16

_CHUNK = 512
_GPIECE = 128


def _pos_signal(length, channels, min_timescale=1.0, max_timescale=10000.0):
    num_timescales = channels // 2
    log_timescale_increment = math.log(
        float(max_timescale) / float(min_timescale)) / (float(num_timescales) - 1.0)
    position = jnp.arange(0, length, dtype=jnp.float32)
    inv_timescales = jnp.exp(
        jnp.arange(0, num_timescales, dtype=jnp.float32)
        * (-log_timescale_increment)) * min_timescale
    scaled_time = position[:, None] * inv_timescales[None, :]
    return jnp.concatenate([jnp.sin(scaled_time), jnp.cos(scaled_time)], axis=1)


def _make_sc_kernel(n_rows, seq_len, dim, scale):
    per_w = n_rows // _NW
    n_chunks = per_w // _CHUNK
    n_pieces = _CHUNK // _GPIECE
    half = _CHUNK // 2
    mesh = plsc.VectorSubcoreMesh(core_axis_name="c", subcore_axis_name="s")

    @functools.partial(
        pl.kernel,
        out_type=jax.ShapeDtypeStruct((n_rows, dim), jnp.float32),
        mesh=mesh,
        compiler_params=pltpu.CompilerParams(use_tc_tiling_on_sc=False),
        scratch_types=[
            pltpu.VMEM((seq_len, dim), jnp.float32),
            pltpu.VMEM((per_w,), jnp.int32),
            pltpu.VMEM((2, _CHUNK, dim), jnp.float32),
            pltpu.SemaphoreType.DMA,
            pltpu.SemaphoreType.DMA,
            pltpu.SemaphoreType.DMA,
            pltpu.SemaphoreType.DMA,
        ],
    )
    def emb_kernel(table_hbm, idx_hbm, sig_hbm, out_hbm,
                   sig_v, idx_v, rows_v, g0, g1, o0, o1):
        cid = lax.axis_index("c")
        sid = lax.axis_index("s")
        wid = sid * _NC + cid
        base = wid * per_w
        gsems = (g0, g1)
        osems = (o0, o1)
        pltpu.sync_copy(sig_hbm, sig_v)
        pltpu.sync_copy(idx_hbm.at[pl.ds(base, per_w)], idx_v)

        def gather_desc(i, b, p, make_only):
            mk = pltpu.make_async_copy if make_only else pltpu.async_copy
            return mk(
                table_hbm.at[idx_v.at[pl.ds(i * _CHUNK + p * _GPIECE, _GPIECE)]],
                rows_v.at[b].at[pl.ds(p * _GPIECE, _GPIECE)],
                gsems[b],
            )

        def out_desc(i, b, make_only):
            mk = pltpu.make_async_copy if make_only else pltpu.async_copy
            return mk(rows_v.at[b], out_hbm.at[pl.ds(base + i * _CHUNK, _CHUNK)],
                      osems[b])

        def compute(i, b):
            @pl.loop(0, _CHUNK, unroll=2)
            def _row(j):
                l = (i * _CHUNK + j) % seq_len
                for d in range(dim // _LANES):
                    sl = pl.ds(d * _LANES, _LANES)
                    rows_v[b, j, sl] = rows_v[b, j, sl] * scale + sig_v[l, sl]

        for p in range(n_pieces):
            gather_desc(0, 0, p, False)

        @pl.loop(0, n_chunks, step=2)
        def _chunk(i):
            for b in range(2):
                cur = i + b
                other = 1 - b

                @pl.when(cur + 1 < n_chunks)
                def _fire_next():
                    @pl.when(cur >= 1)
                    def _wait_out():
                        out_desc(cur - 1, other, True).wait()
                    for p in range(n_pieces):
                        gather_desc(cur + 1, other, p, False)

                for p in range(n_pieces):
                    gather_desc(cur, b, p, True).wait()
                compute(cur, b)
                out_desc(cur, b, False)

        out_desc(n_chunks - 2, (n_chunks - 2) % 2, True).wait()
        out_desc(n_chunks - 1, (n_chunks - 1) % 2, True).wait()

    return emb_kernel


def kernel(x, table):
    b, seq_len = x.shape
    num_emb, dim = table.shape
    scale = float(dim) ** 0.5
    sig = _pos_signal(seq_len, dim)
    xf = x.reshape(-1)
    sc = _make_sc_kernel(b * seq_len, seq_len, dim, scale)
    out = sc(table, xf, sig)
    return out.reshape(b, seq_len, dim)


# speedup vs baseline: 1.2376x; 1.2376x over previous
import functools
import math

import jax
import jax.numpy as jnp
from jax import lax
from jax.experimental import pallas as pl
from jax.experimental.pallas import tpu as pltpu
from jax.experimental.pallas import tpu_sc as plsc

_NC = 2
_NS = 16
_NW = _NC * _NS
_LANES = 16

_CHUNK = 400
_GPIECE = 80


def _pos_signal(length, channels, min_timescale=1.0, max_timescale=10000.0):
    num_timescales = channels // 2
    log_timescale_increment = math.log(
        float(max_timescale) / float(min_timescale)) / (float(num_timescales) - 1.0)
    position = jnp.arange(0, length, dtype=jnp.float32)
    inv_timescales = jnp.exp(
        jnp.arange(0, num_timescales, dtype=jnp.float32)
        * (-log_timescale_increment)) * min_timescale
    scaled_time = position[:, None] * inv_timescales[None, :]
    return jnp.concatenate([jnp.sin(scaled_time), jnp.cos(scaled_time)], axis=1)


def _make_sc_kernel(n_rows, seq_len, dim, scale):
    per_w = n_rows // _NW
    n_chunks = per_w // _CHUNK
    n_pieces = _CHUNK // _GPIECE
    half = _CHUNK // 2
    mesh = plsc.VectorSubcoreMesh(core_axis_name="c", subcore_axis_name="s")

    @functools.partial(
        pl.kernel,
        out_type=jax.ShapeDtypeStruct((n_rows, dim), jnp.float32),
        mesh=mesh,
        compiler_params=pltpu.CompilerParams(use_tc_tiling_on_sc=False),
        scratch_types=[
            pltpu.VMEM((seq_len, dim), jnp.float32),
            pltpu.VMEM((per_w,), jnp.int32),
            pltpu.VMEM((3, _CHUNK, dim), jnp.float32),
            pltpu.SemaphoreType.DMA,
            pltpu.SemaphoreType.DMA,
            pltpu.SemaphoreType.DMA,
            pltpu.SemaphoreType.DMA,
            pltpu.SemaphoreType.DMA,
            pltpu.SemaphoreType.DMA,
        ],
    )
    def emb_kernel(table_hbm, idx_hbm, sig_hbm, out_hbm,
                   sig_v, idx_v, rows_v, g0, g1, g2, o0, o1, o2):
        cid = lax.axis_index("c")
        sid = lax.axis_index("s")
        wid = sid * _NC + cid
        base = wid * per_w
        gsems = (g0, g1, g2)
        osems = (o0, o1, o2)
        pltpu.sync_copy(sig_hbm, sig_v)
        pltpu.sync_copy(idx_hbm.at[pl.ds(base, per_w)], idx_v)

        def gather_desc(i, b, p, make_only):
            mk = pltpu.make_async_copy if make_only else pltpu.async_copy
            return mk(
                table_hbm.at[idx_v.at[pl.ds(i * _CHUNK + p * _GPIECE, _GPIECE)]],
                rows_v.at[b].at[pl.ds(p * _GPIECE, _GPIECE)],
                gsems[b],
            )

        def out_desc(i, b, make_only):
            mk = pltpu.make_async_copy if make_only else pltpu.async_copy
            return mk(rows_v.at[b], out_hbm.at[pl.ds(base + i * _CHUNK, _CHUNK)],
                      osems[b])

        def compute(b):
            @pl.loop(0, half, unroll=2)
            def _row(j):
                for d in range(dim // _LANES):
                    sl = pl.ds(d * _LANES, _LANES)
                    s = sig_v[j, sl]
                    rows_v[b, j, sl] = rows_v[b, j, sl] * scale + s
                    rows_v[b, half + j, sl] = rows_v[b, half + j, sl] * scale + s

        for p in range(n_pieces):
            gather_desc(0, 0, p, False)
        for p in range(n_pieces):
            gather_desc(1, 1, p, False)

        @pl.loop(0, ((n_chunks + 2) // 3) * 3, step=3)
        def _chunk(i):
            for b in range(3):
                cur = i + b
                nxt = (b + 2) % 3

                @pl.when(cur < n_chunks)
                def _body():
                    @pl.when(cur + 2 < n_chunks)
                    def _fire_next():
                        @pl.when(cur >= 1)
                        def _wait_out():
                            out_desc(cur - 1, nxt, True).wait()
                        for p in range(n_pieces):
                            gather_desc(cur + 2, nxt, p, False)

                    for p in range(n_pieces):
                        gather_desc(cur, b, p, True).wait()
                    compute(b)
                    out_desc(cur, b, False)

        out_desc(n_chunks - 3, (n_chunks - 3) % 3, True).wait()
        out_desc(n_chunks - 2, (n_chunks - 2) % 3, True).wait()
        out_desc(n_chunks - 1, (n_chunks - 1) % 3, True).wait()

    return emb_kernel


def kernel(x, table):
    b, seq_len = x.shape
    num_emb, dim = table.shape
    scale = float(dim) ** 0.5
    sig = _pos_signal(seq_len, dim)
    xf = x.reshape(-1)
    sc = _make_sc_kernel(b * seq_len, seq_len, dim, scale)
    out = sc(table, xf, sig)
    return out.reshape(b, seq_len, dim)
